# baseline (speedup 1.0000x reference)
import functools

import jax
import jax.numpy as jnp
from jax import lax
from jax.experimental import pallas as pl
from jax.experimental.pallas import tpu as pltpu
from jax.experimental.pallas import tpu_sc as plsc


def _sc_gather(w_flat, idx, stride):
    B = idx.shape[0]
    info = plsc.get_sparse_core_info()
    nw = info.num_cores * info.num_subcores
    b_per_w = B // nw
    mesh = plsc.VectorSubcoreMesh(core_axis_name="c", subcore_axis_name="s")

    @functools.partial(
        pl.kernel,
        mesh=mesh,
        out_type=jax.ShapeDtypeStruct((B,), jnp.float32),
        scratch_types=[
            pltpu.VMEM((b_per_w,), jnp.int32),
            pltpu.VMEM((b_per_w,), jnp.int32),
            pltpu.VMEM((b_per_w,), jnp.float32),
            pltpu.SemaphoreType.DMA,
        ],
    )
    def gather_kernel(table_hbm, idx_hbm, out_hbm, idx_v, scaled_v, vals_v, sem):
        wid = lax.axis_index("s") * info.num_cores + lax.axis_index("c")
        base = wid * b_per_w
        pltpu.sync_copy(idx_hbm.at[pl.ds(base, b_per_w)], idx_v)
        for k in range(b_per_w // 16):
            sl = pl.ds(k * 16, 16)
            scaled_v[sl] = idx_v[sl] * stride
        pltpu.async_copy(table_hbm.at[scaled_v], vals_v, sem).wait()
        pltpu.sync_copy(vals_v, out_hbm.at[pl.ds(base, b_per_w)])

    return gather_kernel(w_flat, idx)


def _dense_body(x_hbm, y_ref, yfull_ref, out_ref, xbuf, sems, *, n, d, rows,
                bs, grid):
    i = pl.program_id(0)
    slot = jax.lax.rem(i, 2)
    nxt = jax.lax.rem(i + 1, 2)

    @pl.when(i == 0)
    def _():
        pltpu.make_async_copy(x_hbm.at[pl.ds(0, bs)], xbuf.at[0],
                              sems.at[0]).start()

    @pl.when(i + 1 < grid)
    def _():
        pltpu.make_async_copy(x_hbm.at[pl.ds((i + 1) * bs, bs)], xbuf.at[nxt],
                              sems.at[nxt]).start()

    pltpu.make_async_copy(x_hbm.at[pl.ds(i * bs, bs)], xbuf.at[slot],
                          sems.at[slot]).wait()
    x = xbuf[slot]
    y_row = y_ref[0]
    y_sum = jnp.sum(yfull_ref[...])
    t = min(bs, 1024)
    eye = (jax.lax.broadcasted_iota(jnp.int32, (t, t), 0)
           == jax.lax.broadcasted_iota(jnp.int32, (t, t), 1)).astype(jnp.float32)
    cols = [jax.lax.dot_general(eye, y_row[:, k * t:(k + 1) * t],
                                (((1,), (1,)), ((), ())),
                                preferred_element_type=jnp.float32)
            for k in range(bs // t)]
    y = jnp.concatenate(cols, axis=0) if len(cols) > 1 else cols[0]
    s = jnp.sum(x, axis=1, keepdims=True)
    x_avg = s / n
    sx = jnp.sum((x - x_avg) ** 2, axis=1, keepdims=True)
    y_avg = y_sum / n
    sxy = (y - y_avg) * (s - d * x_avg)
    beta = sxy / sx
    pred = beta * s + (y_avg - beta * x_avg)
    out_ref[...] = jnp.broadcast_to(pred, (pred.shape[0], rows))


def _dense(inputs, y, rows, block_samples=4096, interpret=False):
    n, d = inputs.shape
    grid = n // block_samples
    y3 = y.reshape(n // block_samples, 1, block_samples)
    yfull = y.reshape(n // 128, 128)
    return pl.pallas_call(
        functools.partial(_dense_body, n=float(n), d=float(d), rows=rows,
                          bs=block_samples, grid=grid),
        grid=(grid,),
        in_specs=[
            pl.BlockSpec(memory_space=pl.ANY),
            pl.BlockSpec((1, 1, block_samples), lambda i: (i, 0, 0)),
            pl.BlockSpec((n // 128, 128), lambda i: (0, 0)),
        ],
        out_specs=pl.BlockSpec((block_samples, rows), lambda i: (i, 0)),
        out_shape=jax.ShapeDtypeStruct((n, rows), jnp.float32),
        scratch_shapes=[
            pltpu.VMEM((2, block_samples, d), jnp.float32),
            pltpu.SemaphoreType.DMA((2,)),
        ],
        interpret=interpret,
    )(inputs, y3, yfull)


def kernel(inputs, true_label, weights_matrix, training):
    rows = weights_matrix.shape[0]
    w_flat = jnp.transpose(weights_matrix, (1, 2, 0)).reshape(-1)
    y = _sc_gather(w_flat, true_label, rows)
    return _dense(inputs, y, rows)


# speedup vs baseline: 1.0239x; 1.0239x over previous
import functools

import jax
import jax.numpy as jnp
from jax import lax
from jax.experimental import pallas as pl
from jax.experimental.pallas import tpu as pltpu
from jax.experimental.pallas import tpu_sc as plsc


def _sc_gather(w_flat, idx, stride):
    B = idx.shape[0]
    info = plsc.get_sparse_core_info()
    nw = info.num_cores * info.num_subcores
    b_per_w = B // nw
    mesh = plsc.VectorSubcoreMesh(core_axis_name="c", subcore_axis_name="s")

    @functools.partial(
        pl.kernel,
        mesh=mesh,
        out_type=jax.ShapeDtypeStruct((B,), jnp.float32),
        scratch_types=[
            pltpu.VMEM((b_per_w,), jnp.int32),
            pltpu.VMEM((b_per_w,), jnp.int32),
            pltpu.VMEM((b_per_w,), jnp.float32),
            pltpu.SemaphoreType.DMA,
        ],
    )
    def gather_kernel(table_hbm, idx_hbm, out_hbm, idx_v, scaled_v, vals_v, sem):
        wid = lax.axis_index("s") * info.num_cores + lax.axis_index("c")
        base = wid * b_per_w
        pltpu.sync_copy(idx_hbm.at[pl.ds(base, b_per_w)], idx_v)
        for k in range(b_per_w // 16):
            sl = pl.ds(k * 16, 16)
            scaled_v[sl] = idx_v[sl] * stride
        pltpu.async_copy(table_hbm.at[scaled_v], vals_v, sem).wait()
        pltpu.sync_copy(vals_v, out_hbm.at[pl.ds(base, b_per_w)])

    return gather_kernel(w_flat, idx)


def _dense_body(x_ref, y_ref, yfull_ref, out_ref, *, n, d, rows):
    bs = x_ref.shape[0]
    x = x_ref[...]
    y_row = y_ref[0]
    y_sum = jnp.sum(yfull_ref[...])
    t = min(bs, 1024)
    eye = (jax.lax.broadcasted_iota(jnp.int32, (t, t), 0)
           == jax.lax.broadcasted_iota(jnp.int32, (t, t), 1)).astype(jnp.float32)
    cols = [jax.lax.dot_general(eye, y_row[:, k * t:(k + 1) * t],
                                (((1,), (1,)), ((), ())),
                                preferred_element_type=jnp.float32)
            for k in range(bs // t)]
    y = jnp.concatenate(cols, axis=0) if len(cols) > 1 else cols[0]
    s = jnp.sum(x, axis=1, keepdims=True)
    x_avg = s / n
    sx = jnp.sum((x - x_avg) ** 2, axis=1, keepdims=True)
    y_avg = y_sum / n
    sxy = (y - y_avg) * (s - d * x_avg)
    beta = sxy / sx
    pred = beta * s + (y_avg - beta * x_avg)
    out_ref[...] = jnp.broadcast_to(pred, (pred.shape[0], rows))


def _dense(inputs, y, rows, block_samples=4096, interpret=False):
    n, d = inputs.shape
    grid = n // block_samples
    y3 = y.reshape(n // block_samples, 1, block_samples)
    yfull = y.reshape(n // 128, 128)
    return pl.pallas_call(
        functools.partial(_dense_body, n=float(n), d=float(d), rows=rows),
        grid=(grid,),
        in_specs=[
            pl.BlockSpec((block_samples, d), lambda i: (i, 0)),
            pl.BlockSpec((1, 1, block_samples), lambda i: (i, 0, 0)),
            pl.BlockSpec((n // 128, 128), lambda i: (0, 0)),
        ],
        out_specs=pl.BlockSpec((block_samples, rows), lambda i: (i, 0)),
        out_shape=jax.ShapeDtypeStruct((n, rows), jnp.float32),
        interpret=interpret,
    )(inputs, y3, yfull)


def kernel(inputs, true_label, weights_matrix, training):
    rows = weights_matrix.shape[0]
    w_flat = jnp.transpose(weights_matrix, (1, 2, 0)).reshape(-1)
    y = _sc_gather(w_flat, true_label, rows)
    return _dense(inputs, y, rows)
